# baseline (speedup 1.0000x reference)
import functools

import jax
import jax.numpy as jnp
from jax.experimental import pallas as pl
from jax.experimental.pallas import tpu as pltpu

D_MODEL = 768
NUM_EXPERTS = 64
TB = 2048

NEG_BIG = -1e30


def _top2(logits, idx_ref, prob_ref):
    iota = jax.lax.broadcasted_iota(jnp.int32, logits.shape, 1
                                    ).astype(jnp.float32)
    m1 = jnp.max(logits, axis=-1, keepdims=True)
    i1 = jnp.min(jnp.where(logits == m1, iota, float(NUM_EXPERTS)), axis=-1,
                 keepdims=True)
    masked = jnp.where(iota == i1, NEG_BIG, logits)
    m2 = jnp.max(masked, axis=-1, keepdims=True)
    i2 = jnp.min(jnp.where(masked == m2, iota, float(NUM_EXPERTS)), axis=-1,
                 keepdims=True)
    e = jnp.exp(m2 - m1)
    r = 1.0 / (1.0 + e)
    idx_ref[...] = jnp.concatenate([i1, i2], axis=1).astype(jnp.int32)
    prob_ref[...] = jnp.concatenate([r, e * r], axis=1)


def _router_block(xa_ref, xb_ref, w_ref, b_ref,
                  idxa_ref, proba_ref, idxb_ref, probb_ref):
    w = w_ref[...]
    bias = b_ref[...]
    la = jnp.dot(xa_ref[...], w, preferred_element_type=jnp.float32) + bias
    _top2(la, idxa_ref, proba_ref)
    lb = jnp.dot(xb_ref[...], w, preferred_element_type=jnp.float32) + bias
    _top2(lb, idxb_ref, probb_ref)


@functools.partial(jax.jit, static_argnames=())
def kernel(x, W, b):
    n_tokens = x.shape[0]
    half_blocks = n_tokens // (2 * TB)
    b2 = b.reshape(1, NUM_EXPERTS)
    half = n_tokens // 2
    outs = pl.pallas_call(
        _router_block,
        grid=(half_blocks,),
        in_specs=[
            pl.BlockSpec((TB, D_MODEL), lambda i: (i, 0)),
            pl.BlockSpec((TB, D_MODEL),
                         lambda i, hb=half_blocks: (i + hb, 0)),
            pl.BlockSpec((D_MODEL, NUM_EXPERTS), lambda i: (0, 0)),
            pl.BlockSpec((1, NUM_EXPERTS), lambda i: (0, 0)),
        ],
        out_specs=[
            pl.BlockSpec((TB, 2), lambda i: (i, 0)),
            pl.BlockSpec((TB, 2), lambda i: (i, 0)),
            pl.BlockSpec((TB, 2), lambda i: (i, 0)),
            pl.BlockSpec((TB, 2), lambda i: (i, 0)),
        ],
        out_shape=[
            jax.ShapeDtypeStruct((half, 2), jnp.int32),
            jax.ShapeDtypeStruct((half, 2), jnp.float32),
            jax.ShapeDtypeStruct((half, 2), jnp.int32),
            jax.ShapeDtypeStruct((half, 2), jnp.float32),
        ],
        compiler_params=pltpu.CompilerParams(
            dimension_semantics=("arbitrary",),
        ),
    )(x, x, W, b2)
    idx_a, prob_a, idx_b, prob_b = outs
    idx = jnp.concatenate([idx_a, idx_b], axis=0)
    probs = jnp.concatenate([prob_a, prob_b], axis=0)
    return idx, probs


# speedup vs baseline: 1.0907x; 1.0907x over previous
import functools

import jax
import jax.numpy as jnp
from jax.experimental import pallas as pl
from jax.experimental.pallas import tpu as pltpu

D_MODEL = 768
NUM_EXPERTS = 64
TB = 4096

NEG_BIG = -1e30


def _router_block(x_ref, w_ref, b_ref, idx_ref, prob_ref):
    logits = jnp.dot(x_ref[...], w_ref[...],
                     preferred_element_type=jnp.float32) + b_ref[...]

    iota = jax.lax.broadcasted_iota(jnp.int32, logits.shape, 1
                                    ).astype(jnp.float32)
    m1 = jnp.max(logits, axis=-1, keepdims=True)
    i1 = jnp.min(jnp.where(logits == m1, iota, float(NUM_EXPERTS)), axis=-1,
                 keepdims=True)
    masked = jnp.where(iota == i1, NEG_BIG, logits)
    m2 = jnp.max(masked, axis=-1, keepdims=True)
    i2 = jnp.min(jnp.where(masked == m2, iota, float(NUM_EXPERTS)), axis=-1,
                 keepdims=True)

    e = jnp.exp(m2 - m1)
    r = 1.0 / (1.0 + e)
    idx_ref[...] = jnp.concatenate([i1, i2], axis=1).astype(jnp.int32)
    prob_ref[...] = jnp.concatenate([r, e * r], axis=1)


@functools.partial(jax.jit, static_argnames=())
def kernel(x, W, b):
    n_tokens = x.shape[0]
    grid = (n_tokens // TB,)
    b2 = b.reshape(1, NUM_EXPERTS)
    idx, probs = pl.pallas_call(
        _router_block,
        grid=grid,
        in_specs=[
            pl.BlockSpec((TB, D_MODEL), lambda i: (i, 0)),
            pl.BlockSpec((D_MODEL, NUM_EXPERTS), lambda i: (0, 0)),
            pl.BlockSpec((1, NUM_EXPERTS), lambda i: (0, 0)),
        ],
        out_specs=[
            pl.BlockSpec((TB, 2), lambda i: (i, 0)),
            pl.BlockSpec((TB, 2), lambda i: (i, 0)),
        ],
        out_shape=[
            jax.ShapeDtypeStruct((n_tokens, 2), jnp.int32),
            jax.ShapeDtypeStruct((n_tokens, 2), jnp.float32),
        ],
        compiler_params=pltpu.CompilerParams(
            dimension_semantics=("arbitrary",),
        ),
    )(x, W, b2)
    return idx, probs
